# baseline (speedup 1.0000x reference)
import functools

import jax
import jax.numpy as jnp
from jax import lax
from jax.experimental import pallas as pl
from jax.experimental.pallas import tpu as pltpu

N = 1024
H = 128
O = 64
L = 2
TI = 8
TJ = 128
NJT = N // TJ
NIT = N // TI


def _decoder_body(x_ref, AT_ref, Wk_ref, bk_ref, Wq_ref, bq_ref, Wv_ref,
                  bv_ref, Ws_ref, b_ref, mW_ref, mb_ref, lng_ref, lnb_ref,
                  linW_ref, linb_ref, m1W_ref, m1b_ref, m2W_ref, m2b_ref,
                  mu_ref, lv_ref, xs, ks, qs, vs, ss):
    f32 = jnp.float32
    xs[:] = x_ref[:]
    for l in range(L):
        x = xs[:]
        ks[:] = jnp.minimum(jnp.exp(
            -(jnp.dot(x, Wk_ref[l], preferred_element_type=f32) + bk_ref[l:l + 1, :])), 1e30)
        qs[:] = jnp.minimum(jnp.exp(
            -(jnp.dot(x, Wq_ref[l], preferred_element_type=f32) + bq_ref[l:l + 1, :])), 1e30)
        vs[:] = jnp.dot(x, Wv_ref[l], preferred_element_type=f32) + bv_ref[l:l + 1, :]
        ss[:] = jnp.dot(x, Ws_ref[l], preferred_element_type=f32) + b_ref[l:l + 1, :]
        qv = qs[:]
        vv = vs[:]

        def jbody(j, carry):
            ek = ks[pl.ds(j, 1), :]
            d = 1.0 + qv * ek
            r = vv / d
            mrow = AT_ref[pl.ds(j, 1), :]
            aggj = lax.dot_general(mrow, r, (((1,), (0,)), ((), ())),
                                   preferred_element_type=f32)
            xs[pl.ds(j, 1), :] = aggj + ss[pl.ds(j, 1), :]
            return carry

        lax.fori_loop(0, N, jbody, 0)
        h1 = jnp.dot(xs[:], mW_ref[l], preferred_element_type=f32) + mb_ref[l:l + 1, :]
        mu = jnp.mean(h1, axis=-1, keepdims=True)
        var = jnp.mean((h1 - mu) ** 2, axis=-1, keepdims=True)
        hn = (h1 - mu) / jnp.sqrt(var + 1e-5) * lng_ref[l:l + 1, :] + lnb_ref[l:l + 1, :]
        xs[:] = jnp.maximum(hn, 0.0)
    x = xs[:]
    mu_ref[:] = jnp.dot(x, linW_ref[:], preferred_element_type=f32) + linb_ref[0:1, :]
    h = jnp.maximum(jnp.dot(x, m1W_ref[:], preferred_element_type=f32) + m1b_ref[0:1, :], 0.0)
    lv_ref[:] = jnp.dot(h, m2W_ref[:], preferred_element_type=f32) + m2b_ref[0:1, :]


@jax.jit
def _decoder(x, A, Wk, bk, Wq, bq, Wv, bv, Ws, b, mW, mb, lng, lnb,
             linW, linb, m1W, m1b, m2W, m2b):
    mu, lv = pl.pallas_call(
        _decoder_body,
        out_shape=[
            jax.ShapeDtypeStruct((N, O), jnp.float32),
            jax.ShapeDtypeStruct((N, O), jnp.float32),
        ],
        scratch_shapes=[pltpu.VMEM((N, H), jnp.float32)] * 5,
    )(x, A, Wk, bk, Wq, bq, Wv, bv, Ws, b, mW, mb, lng, lnb,
      linW, linb, m1W, m1b, m2W, m2b)
    return mu, lv


def kernel(node_feat, adj, Wk, bk, Wq, bq, Wv, bv, Ws, b, mW, mb, lng, lnb,
           linW, linb, m1W, m1b, m2W, m2b, grad_out=None):
    x = node_feat[0]
    A = adj[0].T
    mu, lv = _decoder(x, A, Wk, bk, Wq, bq, Wv, bv, Ws, b, mW, mb, lng, lnb,
                      linW, linb.reshape(1, O), m1W, m1b.reshape(1, H),
                      m2W, m2b.reshape(1, O))
    return (mu[None], lv[None])


# speedup vs baseline: 1.0913x; 1.0913x over previous
import functools

import jax
import jax.numpy as jnp
from jax import lax
from jax.experimental import pallas as pl
from jax.experimental.pallas import tpu as pltpu

N = 1024
H = 128
O = 64
L = 2
TI = 8
TJ = 128
NJT = N // TJ
NIT = N // TI


def _decoder_body(x_ref, AT_ref, Wk_ref, bk_ref, Wq_ref, bq_ref, Wv_ref,
                  bv_ref, Ws_ref, b_ref, mW_ref, mb_ref, lng_ref, lnb_ref,
                  linW_ref, linb_ref, m1W_ref, m1b_ref, m2W_ref, m2b_ref,
                  mu_ref, lv_ref, xs, ks, qs, vs, ss):
    f32 = jnp.float32
    xs[:] = x_ref[:]
    for l in range(L):
        x = xs[:]
        ks[:] = 0.5 * (jnp.dot(x, Wk_ref[l], preferred_element_type=f32) + bk_ref[l:l + 1, :])
        qs[:] = 0.5 * (jnp.dot(x, Wq_ref[l], preferred_element_type=f32) + bq_ref[l:l + 1, :])
        vs[:] = 0.5 * (jnp.dot(x, Wv_ref[l], preferred_element_type=f32) + bv_ref[l:l + 1, :])
        ss[:] = jnp.dot(x, Ws_ref[l], preferred_element_type=f32) + b_ref[l:l + 1, :]
        for jt in range(NJT):
            kt = ks[jt * TJ:(jt + 1) * TJ, :]

            def ibody(it, acc, kt=kt, jt=jt):
                row = pl.multiple_of(it * TI, TI)
                qt = qs[pl.ds(row, TI), :]
                vt = vs[pl.ds(row, TI), :]
                Mt = AT_ref[pl.ds(row, TI), jt * TJ:(jt + 1) * TJ]
                th = jnp.tanh(kt[None, :, :] + qt[:, None, :])
                vb = vt[:, None, :]
                msg = (vb + vb * th) * Mt[:, :, None]
                return acc + jnp.sum(msg, axis=0)

            agg = lax.fori_loop(0, NIT, ibody, jnp.zeros((TJ, H), f32))
            xs[jt * TJ:(jt + 1) * TJ, :] = agg + ss[jt * TJ:(jt + 1) * TJ, :]
        h1 = jnp.dot(xs[:], mW_ref[l], preferred_element_type=f32) + mb_ref[l:l + 1, :]
        mu = jnp.mean(h1, axis=-1, keepdims=True)
        var = jnp.mean((h1 - mu) ** 2, axis=-1, keepdims=True)
        hn = (h1 - mu) / jnp.sqrt(var + 1e-5) * lng_ref[l:l + 1, :] + lnb_ref[l:l + 1, :]
        xs[:] = jnp.maximum(hn, 0.0)
    x = xs[:]
    mu_ref[:] = jnp.dot(x, linW_ref[:], preferred_element_type=f32) + linb_ref[0:1, :]
    h = jnp.maximum(jnp.dot(x, m1W_ref[:], preferred_element_type=f32) + m1b_ref[0:1, :], 0.0)
    lv_ref[:] = jnp.dot(h, m2W_ref[:], preferred_element_type=f32) + m2b_ref[0:1, :]


@jax.jit
def _decoder(x, A, Wk, bk, Wq, bq, Wv, bv, Ws, b, mW, mb, lng, lnb,
             linW, linb, m1W, m1b, m2W, m2b):
    mu, lv = pl.pallas_call(
        _decoder_body,
        out_shape=[
            jax.ShapeDtypeStruct((N, O), jnp.float32),
            jax.ShapeDtypeStruct((N, O), jnp.float32),
        ],
        scratch_shapes=[pltpu.VMEM((N, H), jnp.float32)] * 5,
    )(x, A, Wk, bk, Wq, bq, Wv, bv, Ws, b, mW, mb, lng, lnb,
      linW, linb, m1W, m1b, m2W, m2b)
    return mu, lv


def kernel(node_feat, adj, Wk, bk, Wq, bq, Wv, bv, Ws, b, mW, mb, lng, lnb,
           linW, linb, m1W, m1b, m2W, m2b, grad_out=None):
    x = node_feat[0]
    A = adj[0]
    mu, lv = _decoder(x, A, Wk, bk, Wq, bq, Wv, bv, Ws, b, mW, mb, lng, lnb,
                      linW, linb.reshape(1, O), m1W, m1b.reshape(1, H),
                      m2W, m2b.reshape(1, O))
    return (mu[None], lv[None])


# speedup vs baseline: 1.2866x; 1.1790x over previous
import functools

import jax
import jax.numpy as jnp
from jax import lax
from jax.experimental import pallas as pl
from jax.experimental.pallas import tpu as pltpu

N = 1024
H = 128
O = 64
L = 2
TI = 16
TJ = 128
NJT = N // TJ
NIT = N // TI


def _decoder_body(x_ref, AT_ref, Wk_ref, bk_ref, Wq_ref, bq_ref, Wv_ref,
                  bv_ref, Ws_ref, b_ref, mW_ref, mb_ref, lng_ref, lnb_ref,
                  linW_ref, linb_ref, m1W_ref, m1b_ref, m2W_ref, m2b_ref,
                  mu_ref, lv_ref, xs, ks, qs, vs, ss):
    f32 = jnp.float32
    xs[:] = x_ref[:]
    for l in range(L):
        x = xs[:]
        ks[:] = 0.5 * (jnp.dot(x, Wk_ref[l], preferred_element_type=f32) + bk_ref[l:l + 1, :])
        qs[:] = 0.5 * (jnp.dot(x, Wq_ref[l], preferred_element_type=f32) + bq_ref[l:l + 1, :])
        vs[:] = 0.5 * (jnp.dot(x, Wv_ref[l], preferred_element_type=f32) + bv_ref[l:l + 1, :])
        ss[:] = jnp.dot(x, Ws_ref[l], preferred_element_type=f32) + b_ref[l:l + 1, :]
        for jt in range(NJT):
            kt = ks[jt * TJ:(jt + 1) * TJ, :]

            def ibody(it, acc, kt=kt, jt=jt):
                row = pl.multiple_of(it * TI, TI)
                qt = qs[pl.ds(row, TI), :]
                vt = vs[pl.ds(row, TI), :]
                Mt = AT_ref[pl.ds(row, TI), jt * TJ:(jt + 1) * TJ]
                th = jnp.tanh(kt[None, :, :] + qt[:, None, :])
                vb = vt[:, None, :]
                msg = (vb + vb * th) * Mt[:, :, None]
                return acc + jnp.sum(msg, axis=0)

            agg = lax.fori_loop(0, NIT, ibody, jnp.zeros((TJ, H), f32))
            xs[jt * TJ:(jt + 1) * TJ, :] = agg + ss[jt * TJ:(jt + 1) * TJ, :]
        h1 = jnp.dot(xs[:], mW_ref[l], preferred_element_type=f32) + mb_ref[l:l + 1, :]
        mu = jnp.mean(h1, axis=-1, keepdims=True)
        var = jnp.mean((h1 - mu) ** 2, axis=-1, keepdims=True)
        hn = (h1 - mu) / jnp.sqrt(var + 1e-5) * lng_ref[l:l + 1, :] + lnb_ref[l:l + 1, :]
        xs[:] = jnp.maximum(hn, 0.0)
    x = xs[:]
    mu_ref[:] = jnp.dot(x, linW_ref[:], preferred_element_type=f32) + linb_ref[0:1, :]
    h = jnp.maximum(jnp.dot(x, m1W_ref[:], preferred_element_type=f32) + m1b_ref[0:1, :], 0.0)
    lv_ref[:] = jnp.dot(h, m2W_ref[:], preferred_element_type=f32) + m2b_ref[0:1, :]


@jax.jit
def _decoder(x, A, Wk, bk, Wq, bq, Wv, bv, Ws, b, mW, mb, lng, lnb,
             linW, linb, m1W, m1b, m2W, m2b):
    mu, lv = pl.pallas_call(
        _decoder_body,
        out_shape=[
            jax.ShapeDtypeStruct((N, O), jnp.float32),
            jax.ShapeDtypeStruct((N, O), jnp.float32),
        ],
        scratch_shapes=[pltpu.VMEM((N, H), jnp.float32)] * 5,
    )(x, A, Wk, bk, Wq, bq, Wv, bv, Ws, b, mW, mb, lng, lnb,
      linW, linb, m1W, m1b, m2W, m2b)
    return mu, lv


def kernel(node_feat, adj, Wk, bk, Wq, bq, Wv, bv, Ws, b, mW, mb, lng, lnb,
           linW, linb, m1W, m1b, m2W, m2b, grad_out=None):
    x = node_feat[0]
    A = adj[0]
    mu, lv = _decoder(x, A, Wk, bk, Wq, bq, Wv, bv, Ws, b, mW, mb, lng, lnb,
                      linW, linb.reshape(1, O), m1W, m1b.reshape(1, H),
                      m2W, m2b.reshape(1, O))
    return (mu[None], lv[None])
